# baseline (speedup 1.0000x reference)
import functools

import jax
import jax.numpy as jnp
from jax import lax
from jax.experimental import pallas as pl
from jax.experimental.pallas import tpu as pltpu
from jax.experimental.pallas import tpu_sc as plsc

SEQ = 8192
DIM = 1024
NUM_CORES = 2
NUM_SUBCORES = 16
NW = NUM_CORES * NUM_SUBCORES
ROWS_PER_W = SEQ // NW
NB = 64
NBLK = ROWS_PER_W // NB


def _pos_embed_sc(batch, idx_hbm, table_hbm, out_hbm, idx_v, rows_v, sem):
    c = lax.axis_index("c")
    s = lax.axis_index("s")
    wid = s * NUM_CORES + c
    base0 = wid * ROWS_PER_W

    def body(i, carry):
        start = base0 + i * NB
        pltpu.sync_copy(idx_hbm.at[pl.ds(start, NB)], idx_v)
        pltpu.async_copy(table_hbm.at[idx_v], rows_v, sem).wait()
        for b in range(batch):
            pltpu.sync_copy(rows_v, out_hbm.at[pl.ds(b * SEQ + start, NB)])
        return carry

    lax.fori_loop(0, NBLK, body, 0)


def kernel(x, emb_weight, offset=0):
    seq = x.shape[-1]
    batch = 1
    for d in x.shape[:-1]:
        batch *= d
    off = jnp.asarray(offset, jnp.int32)
    positions = jnp.clip(jnp.arange(seq, dtype=jnp.int32) + off,
                         0, emb_weight.shape[0] - 1)
    mesh = plsc.VectorSubcoreMesh(core_axis_name="c", subcore_axis_name="s")
    run = pl.kernel(
        functools.partial(_pos_embed_sc, batch),
        mesh=mesh,
        out_type=jax.ShapeDtypeStruct((batch * seq, DIM), jnp.float32),
        scratch_types=[
            pltpu.VMEM((NB,), jnp.int32),
            pltpu.VMEM((NB, DIM), jnp.float32),
            pltpu.SemaphoreType.DMA,
        ],
    )
    out = run(positions, emb_weight)
    return out.reshape(x.shape + (DIM,))
